# baseline (speedup 1.0000x reference)
import functools
import math

import jax
import jax.numpy as jnp
import numpy as np
from jax import lax
from jax.experimental import pallas as pl
from jax.experimental.pallas import tpu as pltpu
from jax.experimental.pallas import tpu_sc as plsc

V = 32000
D = 512
H = 8
NE = 8
DFF = 1024
DH = D // H
_SQRT_D = math.sqrt(D)
_ATTN_SCALE = 1.0 / math.sqrt(DH)
_EMB_ROWS = 16


def _pe_table(S):
    pos = np.arange(S)[:, None].astype(np.float32)
    div = np.exp(np.arange(0, D, 2).astype(np.float32) * (-math.log(10000.0) / D))
    pe = np.zeros((S, D), dtype=np.float32)
    pe[:, 0::2] = np.sin(pos * div)
    pe[:, 1::2] = np.cos(pos * div)
    return jnp.asarray(pe)


def _sc_row_gather(table, idx):
    T = idx.shape[0]
    ch = T // 32
    mesh = plsc.VectorSubcoreMesh(core_axis_name="c", subcore_axis_name="s",
                                  num_cores=2, num_subcores=16)

    def body(table_hbm, idx_hbm, out_hbm, idx_v, rows_v, sem):
        wid = lax.axis_index("s") * 2 + lax.axis_index("c")
        base = wid * ch
        pltpu.sync_copy(idx_hbm.at[pl.ds(base, ch)], idx_v)
        pltpu.async_copy(table_hbm.at[idx_v], rows_v, sem).wait()
        pltpu.sync_copy(rows_v, out_hbm.at[pl.ds(base, ch)])

    f = functools.partial(
        pl.kernel, body, mesh=mesh,
        out_type=jax.ShapeDtypeStruct((T, D), jnp.float32),
        scratch_types=[
            pltpu.VMEM((ch,), jnp.int32),
            pltpu.VMEM((ch, D), jnp.float32),
            pltpu.SemaphoreType.DMA,
        ],
    )
    return f()(table, idx)


def _scale_pe_body(x_ref, pe_ref, o_ref):
    o_ref[...] = x_ref[...] * _SQRT_D + pe_ref[...]


def _embed(table, idx, pe):
    T = idx.shape[0]
    rows = _sc_row_gather(table, idx)
    return pl.pallas_call(
        _scale_pe_body,
        out_shape=jax.ShapeDtypeStruct((T, D), jnp.float32),
    )(rows, pe)


def _mm_bias_body(a_ref, w_ref, b_ref, o_ref):
    o_ref[...] = (
        jnp.dot(a_ref[...], w_ref[...], preferred_element_type=jnp.float32)
        + b_ref[...]
    )


def _mm_bias(a, w, b):
    M, _ = a.shape
    N = w.shape[1]
    return pl.pallas_call(
        _mm_bias_body,
        out_shape=jax.ShapeDtypeStruct((M, N), jnp.float32),
    )(a, w, b.reshape(1, N))


def _mm_bias_blocked(a, w, b, nb):
    M, K = a.shape
    N = w.shape[1]
    return pl.pallas_call(
        _mm_bias_body,
        grid=(N // nb,),
        in_specs=[
            pl.BlockSpec((M, K), lambda j: (0, 0)),
            pl.BlockSpec((K, nb), lambda j: (0, j)),
            pl.BlockSpec((1, nb), lambda j: (0, j)),
        ],
        out_specs=pl.BlockSpec((M, nb), lambda j: (0, j)),
        out_shape=jax.ShapeDtypeStruct((M, N), jnp.float32),
    )(a, w, b.reshape(1, N))


_HP = 4
_NHS = H // _HP


def _attn_one_head(q, k, v):
    s = jax.lax.dot_general(
        q, k, (((1,), (1,)), ((), ())), preferred_element_type=jnp.float32
    ) * _ATTN_SCALE
    m = jnp.max(s, axis=-1, keepdims=True)
    p = jnp.exp(s - m)
    p = p / jnp.sum(p, axis=-1, keepdims=True)
    return jnp.dot(p, v, preferred_element_type=jnp.float32)


def _mha_body(xq_ref, xkv_ref, wq_ref, bq_ref, wk_ref, bk_ref, wv_ref, bv_ref,
              wo_ref, bo_ref, g_ref, bb_ref, o_ref, acc_ref):
    h = pl.program_id(0)
    xq = xq_ref[...]
    xkv = xkv_ref[...]
    q = jnp.dot(xq, wq_ref[0], preferred_element_type=jnp.float32) + bq_ref[0]
    k = jnp.dot(xkv, wk_ref[0], preferred_element_type=jnp.float32) + bk_ref[0]
    v = jnp.dot(xkv, wv_ref[0], preferred_element_type=jnp.float32) + bv_ref[0]
    oh = jnp.concatenate(
        [_attn_one_head(q[:, j * DH:(j + 1) * DH], k[:, j * DH:(j + 1) * DH],
                        v[:, j * DH:(j + 1) * DH]) for j in range(_HP)],
        axis=1,
    )
    contrib = jnp.dot(oh, wo_ref[0], preferred_element_type=jnp.float32)

    @pl.when(h == 0)
    def _():
        acc_ref[...] = contrib

    @pl.when(h != 0)
    def _():
        acc_ref[...] += contrib

    @pl.when(h == _NHS - 1)
    def _():
        t = acc_ref[...] + bo_ref[...] + xq
        mu = jnp.mean(t, axis=-1, keepdims=True)
        var = jnp.mean((t - mu) ** 2, axis=-1, keepdims=True)
        o_ref[...] = (t - mu) * jax.lax.rsqrt(var + 1e-5) * g_ref[...] + bb_ref[...]


def _heads_w(w):
    return w.reshape(D, _NHS, _HP * DH).transpose(1, 0, 2)


def _mha_ln(xq, xkv, ap, lnp):
    Sq = xq.shape[0]
    Skv = xkv.shape[0]
    hw = _HP * DH
    return pl.pallas_call(
        _mha_body,
        grid=(_NHS,),
        in_specs=[
            pl.BlockSpec((Sq, D), lambda h: (0, 0)),
            pl.BlockSpec((Skv, D), lambda h: (0, 0)),
            pl.BlockSpec((1, D, hw), lambda h: (h, 0, 0)),
            pl.BlockSpec((1, 1, hw), lambda h: (h, 0, 0)),
            pl.BlockSpec((1, D, hw), lambda h: (h, 0, 0)),
            pl.BlockSpec((1, 1, hw), lambda h: (h, 0, 0)),
            pl.BlockSpec((1, D, hw), lambda h: (h, 0, 0)),
            pl.BlockSpec((1, 1, hw), lambda h: (h, 0, 0)),
            pl.BlockSpec((1, hw, D), lambda h: (h, 0, 0)),
            pl.BlockSpec((1, D), lambda h: (0, 0)),
            pl.BlockSpec((1, D), lambda h: (0, 0)),
            pl.BlockSpec((1, D), lambda h: (0, 0)),
        ],
        out_specs=pl.BlockSpec((Sq, D), lambda h: (0, 0)),
        out_shape=jax.ShapeDtypeStruct((Sq, D), jnp.float32),
        scratch_shapes=[pltpu.VMEM((Sq, D), jnp.float32)],
    )(
        xq, xkv,
        _heads_w(ap['wq']), ap['bq'].reshape(_NHS, 1, hw),
        _heads_w(ap['wk']), ap['bk'].reshape(_NHS, 1, hw),
        _heads_w(ap['wv']), ap['bv'].reshape(_NHS, 1, hw),
        ap['wo'].reshape(_NHS, hw, D), ap['bo'].reshape(1, D),
        lnp['g'].reshape(1, D), lnp['b'].reshape(1, D),
    )


def _tb_for(T):
    return 512 if T >= 2048 else 256
_RANK_CH = 512


def _router_body(x_ref, rw_ref, rb_ref, dest_ref, wc_ref, be_ref):
    x = x_ref[...]
    T = x.shape[0]
    logits = (
        jnp.dot(x, rw_ref[...], preferred_element_type=jnp.float32) + rb_ref[...]
    )
    mx = jnp.max(logits, axis=-1, keepdims=True)
    ex = jnp.exp(logits - mx)
    probs = ex / jnp.sum(ex, axis=-1, keepdims=True)
    cols = lax.broadcasted_iota(jnp.int32, probs.shape, 1)
    m1 = jnp.max(probs, axis=-1, keepdims=True)
    i1 = jnp.min(jnp.where(probs == m1, cols, NE), axis=-1, keepdims=True)
    masked = jnp.where(cols == i1, -1.0, probs)
    m2 = jnp.max(masked, axis=-1, keepdims=True)
    i2 = jnp.min(jnp.where(masked == m2, cols, NE), axis=-1, keepdims=True)
    s = m1 + m2
    wc_ref[...] = jnp.concatenate([m1 / s, m2 / s], axis=1)

    e_all = jnp.concatenate([i1, i2], axis=0)
    cols2 = lax.broadcasted_iota(jnp.int32, (2 * T, NE), 1)
    onehot = (e_all == cols2).astype(jnp.float32)

    rows_i = lax.broadcasted_iota(jnp.int32, (_RANK_CH, _RANK_CH), 0)
    cols_i = lax.broadcasted_iota(jnp.int32, (_RANK_CH, _RANK_CH), 1)
    tril = (rows_i > cols_i).astype(jnp.float32)
    ranks = []
    base = jnp.zeros((1, NE), jnp.float32)
    for c in range(2 * T // _RANK_CH):
        oh = onehot[c * _RANK_CH:(c + 1) * _RANK_CH]
        pc = jnp.dot(tril, oh, preferred_element_type=jnp.float32) + base
        ranks.append(jnp.sum(pc * oh, axis=1, keepdims=True))
        base = base + jnp.sum(oh, axis=0, keepdims=True)
    rank = jnp.concatenate(ranks, axis=0)
    cnt = base
    tb = _tb_for(T)
    padded = jnp.floor((cnt + (tb - 1)) * (1.0 / tb)).astype(jnp.int32) * tb
    er = lax.broadcasted_iota(jnp.int32, (NE, NE), 0)
    ec = lax.broadcasted_iota(jnp.int32, (NE, NE), 1)
    triu = (er < ec).astype(jnp.float32)
    off = jnp.dot(padded.astype(jnp.float32), triu,
                  preferred_element_type=jnp.float32)
    off_a = jnp.dot(onehot, off.reshape(NE, 1),
                    preferred_element_type=jnp.float32)
    dest_ref[...] = (off_a + rank + 0.5).astype(jnp.int32)

    NB = be_ref.shape[0]
    gstart = lax.broadcasted_iota(jnp.int32, (NB, NE), 0) * tb
    be_ref[...] = jnp.sum(
        (gstart >= off.astype(jnp.int32)).astype(jnp.int32),
        axis=1, keepdims=True) - 1


def _router(x, mp, nb):
    T = x.shape[0]
    return pl.pallas_call(
        _router_body,
        out_shape=(
            jax.ShapeDtypeStruct((2 * T, 1), jnp.int32),
            jax.ShapeDtypeStruct((T, 2), jnp.float32),
            jax.ShapeDtypeStruct((nb, 1), jnp.int32),
        ),
    )(x, mp['rw'], mp['rb'].reshape(1, NE))


def _sc_scatter_rows(x, dest, gtot):
    T = x.shape[0]
    ch = 2 * T // 32
    mesh = plsc.VectorSubcoreMesh(core_axis_name="c", subcore_axis_name="s", num_cores=2, num_subcores=16)

    def body(x_hbm, dest_hbm, out_hbm, idx_v, rows_v, sem):
        wid = lax.axis_index("s") * 2 + lax.axis_index("c")
        base = wid * ch
        tok = base % T
        pltpu.sync_copy(dest_hbm.at[pl.ds(base, ch)], idx_v)
        pltpu.sync_copy(x_hbm.at[pl.ds(tok, ch)], rows_v)
        pltpu.async_copy(rows_v, out_hbm.at[idx_v], sem).wait()

    f = functools.partial(
        pl.kernel, body, mesh=mesh,
        out_type=jax.ShapeDtypeStruct((gtot, D), jnp.float32),
        scratch_types=[
            pltpu.VMEM((ch,), jnp.int32),
            pltpu.VMEM((ch, D), jnp.float32),
            pltpu.SemaphoreType.DMA,
        ],
    )
    return f()(x, dest)


def _sc_gather_rows(sy, dest, ntot):
    ch = ntot // 32
    mesh = plsc.VectorSubcoreMesh(core_axis_name="c", subcore_axis_name="s", num_cores=2, num_subcores=16)

    def body(sy_hbm, dest_hbm, out_hbm, idx_v, rows_v, sem):
        wid = lax.axis_index("s") * 2 + lax.axis_index("c")
        base = wid * ch
        pltpu.sync_copy(dest_hbm.at[pl.ds(base, ch)], idx_v)
        pltpu.async_copy(sy_hbm.at[idx_v], rows_v, sem).wait()
        pltpu.sync_copy(rows_v, out_hbm.at[pl.ds(base, ch)])

    f = functools.partial(
        pl.kernel, body, mesh=mesh,
        out_type=jax.ShapeDtypeStruct((ntot, D), jnp.float32),
        scratch_types=[
            pltpu.VMEM((ch,), jnp.int32),
            pltpu.VMEM((ch, D), jnp.float32),
            pltpu.SemaphoreType.DMA,
        ],
    )
    return f()(sy, dest)


def _ffn_body(be_ref, x_ref, w1_ref, b1_ref, w2_ref, b2_ref, o_ref):
    del be_ref
    h = jnp.maximum(
        jnp.dot(x_ref[...], w1_ref[0], preferred_element_type=jnp.float32)
        + b1_ref[0],
        0.0,
    )
    o_ref[...] = (
        jnp.dot(h, w2_ref[0], preferred_element_type=jnp.float32) + b2_ref[0]
    )


def _grouped_ffn(sx, be, mp, nb, tb):
    spec = pltpu.PrefetchScalarGridSpec(
        num_scalar_prefetch=1,
        grid=(nb,),
        in_specs=[
            pl.BlockSpec((tb, D), lambda g, be: (g, 0)),
            pl.BlockSpec((1, D, DFF), lambda g, be: (be[g], 0, 0)),
            pl.BlockSpec((1, 1, DFF), lambda g, be: (be[g], 0, 0)),
            pl.BlockSpec((1, DFF, D), lambda g, be: (be[g], 0, 0)),
            pl.BlockSpec((1, 1, D), lambda g, be: (be[g], 0, 0)),
        ],
        out_specs=pl.BlockSpec((tb, D), lambda g, be: (g, 0)),
    )
    return pl.pallas_call(
        _ffn_body,
        grid_spec=spec,
        out_shape=jax.ShapeDtypeStruct((nb * tb, D), jnp.float32),
    )(be, sx, mp['w1'], mp['b1'].reshape(NE, 1, DFF), mp['w2'],
      mp['b2'].reshape(NE, 1, D))


def _combine_body(x_ref, y0_ref, y1_ref, wc_ref, g_ref, bb_ref, o_ref):
    wc = wc_ref[...]
    t = x_ref[...] + wc[:, 0:1] * y0_ref[...] + wc[:, 1:2] * y1_ref[...]
    mu = jnp.mean(t, axis=-1, keepdims=True)
    var = jnp.mean((t - mu) ** 2, axis=-1, keepdims=True)
    o_ref[...] = (t - mu) * jax.lax.rsqrt(var + 1e-5) * g_ref[...] + bb_ref[...]


def _combine_ln(x, yall, wc, g, beta):
    T = x.shape[0]
    return pl.pallas_call(
        _combine_body,
        out_shape=jax.ShapeDtypeStruct((T, D), jnp.float32),
    )(x, yall[:T], yall[T:], wc, g.reshape(1, D), beta.reshape(1, D))


def _moe_ln(x, mp, g, beta):
    T = x.shape[0]
    tb = _tb_for(T)
    nb = 2 * T // tb + NE
    dest, wc, be = _router(x, mp, nb)
    dest1 = dest.reshape(2 * T)
    sx = _sc_scatter_rows(x, dest1, nb * tb)
    sy = _grouped_ffn(sx, be.reshape(nb), mp, nb, tb)
    yall = _sc_gather_rows(sy, dest1, 2 * T)
    return _combine_ln(x, yall, wc, g, beta)


def kernel(src, tgt, params):
    p = params
    src_i = src[0]
    tgt_i = tgt[0]

    x = _embed(p['enc_emb'], src_i, _pe_table(src_i.shape[0]))
    for lp in p['enc']:
        x = _mha_ln(x, x, lp['sa'], lp['ln1'])
        x = _moe_ln(x, lp['moe'], lp['ln2']['g'], lp['ln2']['b'])
    mem = x

    y = _embed(p['dec_emb'], tgt_i, _pe_table(tgt_i.shape[0]))
    for lp in p['dec']:
        y = _mha_ln(y, y, lp['sa'], lp['ln1'])
        y = _mha_ln(y, mem, lp['ca'], lp['ln2'])
        y = _moe_ln(y, lp['moe'], lp['ln3']['g'], lp['ln3']['b'])

    logits = _mm_bias_blocked(y, p['out_w'], p['out_b'], 3200)
    return logits[None]


# speedup vs baseline: 1.0722x; 1.0722x over previous
import functools
import math

import jax
import jax.numpy as jnp
import numpy as np
from jax import lax
from jax.experimental import pallas as pl
from jax.experimental.pallas import tpu as pltpu
from jax.experimental.pallas import tpu_sc as plsc

V = 32000
D = 512
H = 8
NE = 8
DFF = 1024
DH = D // H
_SQRT_D = math.sqrt(D)
_ATTN_SCALE = 1.0 / math.sqrt(DH)
_EMB_ROWS = 16


def _pe_table(S):
    pos = np.arange(S)[:, None].astype(np.float32)
    div = np.exp(np.arange(0, D, 2).astype(np.float32) * (-math.log(10000.0) / D))
    pe = np.zeros((S, D), dtype=np.float32)
    pe[:, 0::2] = np.sin(pos * div)
    pe[:, 1::2] = np.cos(pos * div)
    return jnp.asarray(pe)


def _sc_row_gather(table, idx):
    T = idx.shape[0]
    ch = T // 32
    mesh = plsc.VectorSubcoreMesh(core_axis_name="c", subcore_axis_name="s",
                                  num_cores=2, num_subcores=16)

    def body(table_hbm, idx_hbm, out_hbm, idx_v, rows_v, sem):
        wid = lax.axis_index("s") * 2 + lax.axis_index("c")
        base = wid * ch
        pltpu.sync_copy(idx_hbm.at[pl.ds(base, ch)], idx_v)
        pltpu.async_copy(table_hbm.at[idx_v], rows_v, sem).wait()
        pltpu.sync_copy(rows_v, out_hbm.at[pl.ds(base, ch)])

    f = functools.partial(
        pl.kernel, body, mesh=mesh,
        out_type=jax.ShapeDtypeStruct((T, D), jnp.float32),
        scratch_types=[
            pltpu.VMEM((ch,), jnp.int32),
            pltpu.VMEM((ch, D), jnp.float32),
            pltpu.SemaphoreType.DMA,
        ],
    )
    return f()(table, idx)


def _scale_pe_body(x_ref, pe_ref, o_ref):
    o_ref[...] = x_ref[...] * _SQRT_D + pe_ref[...]


def _embed(table, idx, pe):
    T = idx.shape[0]
    rows = _sc_row_gather(table, idx)
    return pl.pallas_call(
        _scale_pe_body,
        out_shape=jax.ShapeDtypeStruct((T, D), jnp.float32),
    )(rows, pe)


def _mm_bias_body(a_ref, w_ref, b_ref, o_ref):
    o_ref[...] = (
        jnp.dot(a_ref[...], w_ref[...], preferred_element_type=jnp.float32)
        + b_ref[...]
    )


def _mm_bias(a, w, b):
    M, _ = a.shape
    N = w.shape[1]
    return pl.pallas_call(
        _mm_bias_body,
        out_shape=jax.ShapeDtypeStruct((M, N), jnp.float32),
    )(a, w, b.reshape(1, N))


def _mm_bias_blocked(a, w, b, nb):
    M, K = a.shape
    N = w.shape[1]
    return pl.pallas_call(
        _mm_bias_body,
        grid=(N // nb,),
        in_specs=[
            pl.BlockSpec((M, K), lambda j: (0, 0)),
            pl.BlockSpec((K, nb), lambda j: (0, j)),
            pl.BlockSpec((1, nb), lambda j: (0, j)),
        ],
        out_specs=pl.BlockSpec((M, nb), lambda j: (0, j)),
        out_shape=jax.ShapeDtypeStruct((M, N), jnp.float32),
    )(a, w, b.reshape(1, N))


_HP = 2
_NHS = H // _HP


def _attn_one_head(q, k, v):
    s = jax.lax.dot_general(
        q, k, (((1,), (1,)), ((), ())), preferred_element_type=jnp.float32
    ) * _ATTN_SCALE
    m = jnp.max(s, axis=-1, keepdims=True)
    p = jnp.exp(s - m)
    p = p / jnp.sum(p, axis=-1, keepdims=True)
    return jnp.dot(p, v, preferred_element_type=jnp.float32)


def _mha_body(xq_ref, xkv_ref, wq_ref, bq_ref, wk_ref, bk_ref, wv_ref, bv_ref,
              wo_ref, bo_ref, g_ref, bb_ref, o_ref, acc_ref):
    h = pl.program_id(0)
    xq = xq_ref[...]
    xkv = xkv_ref[...]
    q = jnp.dot(xq, wq_ref[0], preferred_element_type=jnp.float32) + bq_ref[0]
    k = jnp.dot(xkv, wk_ref[0], preferred_element_type=jnp.float32) + bk_ref[0]
    v = jnp.dot(xkv, wv_ref[0], preferred_element_type=jnp.float32) + bv_ref[0]
    oh = jnp.concatenate(
        [_attn_one_head(q[:, j * DH:(j + 1) * DH], k[:, j * DH:(j + 1) * DH],
                        v[:, j * DH:(j + 1) * DH]) for j in range(_HP)],
        axis=1,
    )
    contrib = jnp.dot(oh, wo_ref[0], preferred_element_type=jnp.float32)

    @pl.when(h == 0)
    def _():
        acc_ref[...] = contrib

    @pl.when(h != 0)
    def _():
        acc_ref[...] += contrib

    @pl.when(h == _NHS - 1)
    def _():
        t = acc_ref[...] + bo_ref[...] + xq
        mu = jnp.mean(t, axis=-1, keepdims=True)
        var = jnp.mean((t - mu) ** 2, axis=-1, keepdims=True)
        o_ref[...] = (t - mu) * jax.lax.rsqrt(var + 1e-5) * g_ref[...] + bb_ref[...]


def _heads_w(w):
    return w.reshape(D, _NHS, _HP * DH).transpose(1, 0, 2)


def _mha_ln(xq, xkv, ap, lnp):
    Sq = xq.shape[0]
    Skv = xkv.shape[0]
    hw = _HP * DH
    return pl.pallas_call(
        _mha_body,
        grid=(_NHS,),
        in_specs=[
            pl.BlockSpec((Sq, D), lambda h: (0, 0)),
            pl.BlockSpec((Skv, D), lambda h: (0, 0)),
            pl.BlockSpec((1, D, hw), lambda h: (h, 0, 0)),
            pl.BlockSpec((1, 1, hw), lambda h: (h, 0, 0)),
            pl.BlockSpec((1, D, hw), lambda h: (h, 0, 0)),
            pl.BlockSpec((1, 1, hw), lambda h: (h, 0, 0)),
            pl.BlockSpec((1, D, hw), lambda h: (h, 0, 0)),
            pl.BlockSpec((1, 1, hw), lambda h: (h, 0, 0)),
            pl.BlockSpec((1, hw, D), lambda h: (h, 0, 0)),
            pl.BlockSpec((1, D), lambda h: (0, 0)),
            pl.BlockSpec((1, D), lambda h: (0, 0)),
            pl.BlockSpec((1, D), lambda h: (0, 0)),
        ],
        out_specs=pl.BlockSpec((Sq, D), lambda h: (0, 0)),
        out_shape=jax.ShapeDtypeStruct((Sq, D), jnp.float32),
        scratch_shapes=[pltpu.VMEM((Sq, D), jnp.float32)],
    )(
        xq, xkv,
        _heads_w(ap['wq']), ap['bq'].reshape(_NHS, 1, hw),
        _heads_w(ap['wk']), ap['bk'].reshape(_NHS, 1, hw),
        _heads_w(ap['wv']), ap['bv'].reshape(_NHS, 1, hw),
        ap['wo'].reshape(_NHS, hw, D), ap['bo'].reshape(1, D),
        lnp['g'].reshape(1, D), lnp['b'].reshape(1, D),
    )


def _tb_for(T):
    return 512 if T >= 2048 else 256
_RANK_CH = 512


def _router_body(x_ref, rw_ref, rb_ref, dest_ref, wc_ref, be_ref):
    x = x_ref[...]
    T = x.shape[0]
    logits = (
        jnp.dot(x, rw_ref[...], preferred_element_type=jnp.float32) + rb_ref[...]
    )
    mx = jnp.max(logits, axis=-1, keepdims=True)
    ex = jnp.exp(logits - mx)
    probs = ex / jnp.sum(ex, axis=-1, keepdims=True)
    cols = lax.broadcasted_iota(jnp.int32, probs.shape, 1)
    m1 = jnp.max(probs, axis=-1, keepdims=True)
    i1 = jnp.min(jnp.where(probs == m1, cols, NE), axis=-1, keepdims=True)
    masked = jnp.where(cols == i1, -1.0, probs)
    m2 = jnp.max(masked, axis=-1, keepdims=True)
    i2 = jnp.min(jnp.where(masked == m2, cols, NE), axis=-1, keepdims=True)
    s = m1 + m2
    wc_ref[...] = jnp.concatenate([m1 / s, m2 / s], axis=1)

    e_all = jnp.concatenate([i1, i2], axis=0)
    cols2 = lax.broadcasted_iota(jnp.int32, (2 * T, NE), 1)
    onehot = (e_all == cols2).astype(jnp.float32)

    rows_i = lax.broadcasted_iota(jnp.int32, (_RANK_CH, _RANK_CH), 0)
    cols_i = lax.broadcasted_iota(jnp.int32, (_RANK_CH, _RANK_CH), 1)
    tril = (rows_i > cols_i).astype(jnp.float32)
    ranks = []
    base = jnp.zeros((1, NE), jnp.float32)
    for c in range(2 * T // _RANK_CH):
        oh = onehot[c * _RANK_CH:(c + 1) * _RANK_CH]
        pc = jnp.dot(tril, oh, preferred_element_type=jnp.float32) + base
        ranks.append(jnp.sum(pc * oh, axis=1, keepdims=True))
        base = base + jnp.sum(oh, axis=0, keepdims=True)
    rank = jnp.concatenate(ranks, axis=0)
    cnt = base
    tb = _tb_for(T)
    padded = jnp.floor((cnt + (tb - 1)) * (1.0 / tb)).astype(jnp.int32) * tb
    er = lax.broadcasted_iota(jnp.int32, (NE, NE), 0)
    ec = lax.broadcasted_iota(jnp.int32, (NE, NE), 1)
    triu = (er < ec).astype(jnp.float32)
    off = jnp.dot(padded.astype(jnp.float32), triu,
                  preferred_element_type=jnp.float32)
    off_a = jnp.dot(onehot, off.reshape(NE, 1),
                    preferred_element_type=jnp.float32)
    dest_ref[...] = (off_a + rank + 0.5).astype(jnp.int32)

    NB = be_ref.shape[0]
    gstart = lax.broadcasted_iota(jnp.int32, (NB, NE), 0) * tb
    be_ref[...] = jnp.sum(
        (gstart >= off.astype(jnp.int32)).astype(jnp.int32),
        axis=1, keepdims=True) - 1


def _router(x, mp, nb):
    T = x.shape[0]
    return pl.pallas_call(
        _router_body,
        out_shape=(
            jax.ShapeDtypeStruct((2 * T, 1), jnp.int32),
            jax.ShapeDtypeStruct((T, 2), jnp.float32),
            jax.ShapeDtypeStruct((nb, 1), jnp.int32),
        ),
    )(x, mp['rw'], mp['rb'].reshape(1, NE))


def _sc_scatter_rows(x, dest, gtot):
    T = x.shape[0]
    ch = 2 * T // 32
    mesh = plsc.VectorSubcoreMesh(core_axis_name="c", subcore_axis_name="s", num_cores=2, num_subcores=16)

    def body(x_hbm, dest_hbm, out_hbm, idx_v, rows_v, sem):
        wid = lax.axis_index("s") * 2 + lax.axis_index("c")
        base = wid * ch
        tok = base % T
        pltpu.sync_copy(dest_hbm.at[pl.ds(base, ch)], idx_v)
        pltpu.sync_copy(x_hbm.at[pl.ds(tok, ch)], rows_v)
        pltpu.async_copy(rows_v, out_hbm.at[idx_v], sem).wait()

    f = functools.partial(
        pl.kernel, body, mesh=mesh,
        out_type=jax.ShapeDtypeStruct((gtot, D), jnp.float32),
        scratch_types=[
            pltpu.VMEM((ch,), jnp.int32),
            pltpu.VMEM((ch, D), jnp.float32),
            pltpu.SemaphoreType.DMA,
        ],
    )
    return f()(x, dest)


def _sc_gather_rows(sy, dest, ntot):
    ch = ntot // 32
    mesh = plsc.VectorSubcoreMesh(core_axis_name="c", subcore_axis_name="s", num_cores=2, num_subcores=16)

    def body(sy_hbm, dest_hbm, out_hbm, idx_v, rows_v, sem):
        wid = lax.axis_index("s") * 2 + lax.axis_index("c")
        base = wid * ch
        pltpu.sync_copy(dest_hbm.at[pl.ds(base, ch)], idx_v)
        pltpu.async_copy(sy_hbm.at[idx_v], rows_v, sem).wait()
        pltpu.sync_copy(rows_v, out_hbm.at[pl.ds(base, ch)])

    f = functools.partial(
        pl.kernel, body, mesh=mesh,
        out_type=jax.ShapeDtypeStruct((ntot, D), jnp.float32),
        scratch_types=[
            pltpu.VMEM((ch,), jnp.int32),
            pltpu.VMEM((ch, D), jnp.float32),
            pltpu.SemaphoreType.DMA,
        ],
    )
    return f()(sy, dest)


def _ffn_body(be_ref, x_ref, w1_ref, b1_ref, w2_ref, b2_ref, o_ref):
    del be_ref
    h = jnp.maximum(
        jnp.dot(x_ref[...], w1_ref[0], preferred_element_type=jnp.float32)
        + b1_ref[0],
        0.0,
    )
    o_ref[...] = (
        jnp.dot(h, w2_ref[0], preferred_element_type=jnp.float32) + b2_ref[0]
    )


def _grouped_ffn(sx, be, mp, nb, tb):
    spec = pltpu.PrefetchScalarGridSpec(
        num_scalar_prefetch=1,
        grid=(nb,),
        in_specs=[
            pl.BlockSpec((tb, D), lambda g, be: (g, 0)),
            pl.BlockSpec((1, D, DFF), lambda g, be: (be[g], 0, 0)),
            pl.BlockSpec((1, 1, DFF), lambda g, be: (be[g], 0, 0)),
            pl.BlockSpec((1, DFF, D), lambda g, be: (be[g], 0, 0)),
            pl.BlockSpec((1, 1, D), lambda g, be: (be[g], 0, 0)),
        ],
        out_specs=pl.BlockSpec((tb, D), lambda g, be: (g, 0)),
    )
    return pl.pallas_call(
        _ffn_body,
        grid_spec=spec,
        out_shape=jax.ShapeDtypeStruct((nb * tb, D), jnp.float32),
    )(be, sx, mp['w1'], mp['b1'].reshape(NE, 1, DFF), mp['w2'],
      mp['b2'].reshape(NE, 1, D))


def _combine_body(x_ref, y0_ref, y1_ref, wc_ref, g_ref, bb_ref, o_ref):
    wc = wc_ref[...]
    t = x_ref[...] + wc[:, 0:1] * y0_ref[...] + wc[:, 1:2] * y1_ref[...]
    mu = jnp.mean(t, axis=-1, keepdims=True)
    var = jnp.mean((t - mu) ** 2, axis=-1, keepdims=True)
    o_ref[...] = (t - mu) * jax.lax.rsqrt(var + 1e-5) * g_ref[...] + bb_ref[...]


def _combine_ln(x, yall, wc, g, beta):
    T = x.shape[0]
    return pl.pallas_call(
        _combine_body,
        out_shape=jax.ShapeDtypeStruct((T, D), jnp.float32),
    )(x, yall[:T], yall[T:], wc, g.reshape(1, D), beta.reshape(1, D))


def _moe_ln(x, mp, g, beta):
    T = x.shape[0]
    tb = _tb_for(T)
    nb = 2 * T // tb + NE
    dest, wc, be = _router(x, mp, nb)
    dest1 = dest.reshape(2 * T)
    sx = _sc_scatter_rows(x, dest1, nb * tb)
    sy = _grouped_ffn(sx, be.reshape(nb), mp, nb, tb)
    yall = _sc_gather_rows(sy, dest1, 2 * T)
    return _combine_ln(x, yall, wc, g, beta)


def kernel(src, tgt, params):
    p = params
    src_i = src[0]
    tgt_i = tgt[0]

    x = _embed(p['enc_emb'], src_i, _pe_table(src_i.shape[0]))
    for lp in p['enc']:
        x = _mha_ln(x, x, lp['sa'], lp['ln1'])
        x = _moe_ln(x, lp['moe'], lp['ln2']['g'], lp['ln2']['b'])
    mem = x

    y = _embed(p['dec_emb'], tgt_i, _pe_table(tgt_i.shape[0]))
    for lp in p['dec']:
        y = _mha_ln(y, y, lp['sa'], lp['ln1'])
        y = _mha_ln(y, mem, lp['ca'], lp['ln2'])
        y = _moe_ln(y, lp['moe'], lp['ln3']['g'], lp['ln3']['b'])

    logits = _mm_bias_blocked(y, p['out_w'], p['out_b'], 3200)
    return logits[None]


# speedup vs baseline: 1.1236x; 1.0479x over previous
import functools
import math

import jax
import jax.numpy as jnp
import numpy as np
from jax import lax
from jax.experimental import pallas as pl
from jax.experimental.pallas import tpu as pltpu
from jax.experimental.pallas import tpu_sc as plsc

V = 32000
D = 512
H = 8
NE = 8
DFF = 1024
DH = D // H
_SQRT_D = math.sqrt(D)
_ATTN_SCALE = 1.0 / math.sqrt(DH)
_EMB_ROWS = 16


def _pe_table(S):
    pos = np.arange(S)[:, None].astype(np.float32)
    div = np.exp(np.arange(0, D, 2).astype(np.float32) * (-math.log(10000.0) / D))
    pe = np.zeros((S, D), dtype=np.float32)
    pe[:, 0::2] = np.sin(pos * div)
    pe[:, 1::2] = np.cos(pos * div)
    return jnp.asarray(pe)


def _sc_row_gather(table, idx):
    T = idx.shape[0]
    ch = T // 32
    mesh = plsc.VectorSubcoreMesh(core_axis_name="c", subcore_axis_name="s",
                                  num_cores=2, num_subcores=16)

    def body(table_hbm, idx_hbm, out_hbm, idx_v, rows_v, sem):
        wid = lax.axis_index("s") * 2 + lax.axis_index("c")
        base = wid * ch
        pltpu.sync_copy(idx_hbm.at[pl.ds(base, ch)], idx_v)
        pltpu.async_copy(table_hbm.at[idx_v], rows_v, sem).wait()
        pltpu.sync_copy(rows_v, out_hbm.at[pl.ds(base, ch)])

    f = functools.partial(
        pl.kernel, body, mesh=mesh,
        out_type=jax.ShapeDtypeStruct((T, D), jnp.float32),
        scratch_types=[
            pltpu.VMEM((ch,), jnp.int32),
            pltpu.VMEM((ch, D), jnp.float32),
            pltpu.SemaphoreType.DMA,
        ],
    )
    return f()(table, idx)


def _scale_pe_body(x_ref, pe_ref, o_ref):
    o_ref[...] = x_ref[...] * _SQRT_D + pe_ref[...]


def _embed(table, idx, pe):
    T = idx.shape[0]
    rows = _sc_row_gather(table, idx)
    return pl.pallas_call(
        _scale_pe_body,
        out_shape=jax.ShapeDtypeStruct((T, D), jnp.float32),
    )(rows, pe)


def _mm_bias_body(a_ref, w_ref, b_ref, o_ref):
    o_ref[...] = (
        jnp.dot(a_ref[...], w_ref[...], preferred_element_type=jnp.float32)
        + b_ref[...]
    )


def _mm_bias(a, w, b):
    M, _ = a.shape
    N = w.shape[1]
    return pl.pallas_call(
        _mm_bias_body,
        out_shape=jax.ShapeDtypeStruct((M, N), jnp.float32),
    )(a, w, b.reshape(1, N))


def _mm_bias_blocked(a, w, b, nb):
    M, K = a.shape
    N = w.shape[1]
    return pl.pallas_call(
        _mm_bias_body,
        grid=(N // nb,),
        in_specs=[
            pl.BlockSpec((M, K), lambda j: (0, 0)),
            pl.BlockSpec((K, nb), lambda j: (0, j)),
            pl.BlockSpec((1, nb), lambda j: (0, j)),
        ],
        out_specs=pl.BlockSpec((M, nb), lambda j: (0, j)),
        out_shape=jax.ShapeDtypeStruct((M, N), jnp.float32),
    )(a, w, b.reshape(1, N))


_HP = 2
_NHS = H // _HP


def _attn_one_head(q, k, v):
    s = jax.lax.dot_general(
        q, k, (((1,), (1,)), ((), ())), preferred_element_type=jnp.float32
    ) * _ATTN_SCALE
    m = jnp.max(s, axis=-1, keepdims=True)
    p = jnp.exp(s - m)
    r = jnp.sum(p, axis=-1, keepdims=True)
    pb = p.astype(jnp.bfloat16)
    return jnp.dot(pb, v, preferred_element_type=jnp.float32) * (1.0 / r)


def _mha_body(xq_ref, xkv_ref, wq_ref, bq_ref, wk_ref, bk_ref, wv_ref, bv_ref,
              wo_ref, bo_ref, g_ref, bb_ref, o_ref, acc_ref):
    h = pl.program_id(0)
    xq = xq_ref[...]
    xkv = xkv_ref[...]
    q = jnp.dot(xq, wq_ref[0], preferred_element_type=jnp.float32) + bq_ref[0]
    k = jnp.dot(xkv, wk_ref[0], preferred_element_type=jnp.float32) + bk_ref[0]
    v = jnp.dot(xkv, wv_ref[0], preferred_element_type=jnp.float32) + bv_ref[0]
    oh = jnp.concatenate(
        [_attn_one_head(q[:, j * DH:(j + 1) * DH], k[:, j * DH:(j + 1) * DH],
                        v[:, j * DH:(j + 1) * DH]) for j in range(_HP)],
        axis=1,
    )
    contrib = jnp.dot(oh, wo_ref[0], preferred_element_type=jnp.float32)

    @pl.when(h == 0)
    def _():
        acc_ref[...] = contrib

    @pl.when(h != 0)
    def _():
        acc_ref[...] += contrib

    @pl.when(h == _NHS - 1)
    def _():
        t = acc_ref[...] + bo_ref[...] + xq
        mu = jnp.mean(t, axis=-1, keepdims=True)
        var = jnp.mean((t - mu) ** 2, axis=-1, keepdims=True)
        o_ref[...] = (t - mu) * jax.lax.rsqrt(var + 1e-5) * g_ref[...] + bb_ref[...]


def _heads_w(w):
    return w.reshape(D, _NHS, _HP * DH).transpose(1, 0, 2)


def _mha_ln(xq, xkv, ap, lnp):
    Sq = xq.shape[0]
    Skv = xkv.shape[0]
    hw = _HP * DH
    return pl.pallas_call(
        _mha_body,
        grid=(_NHS,),
        in_specs=[
            pl.BlockSpec((Sq, D), lambda h: (0, 0)),
            pl.BlockSpec((Skv, D), lambda h: (0, 0)),
            pl.BlockSpec((1, D, hw), lambda h: (h, 0, 0)),
            pl.BlockSpec((1, 1, hw), lambda h: (h, 0, 0)),
            pl.BlockSpec((1, D, hw), lambda h: (h, 0, 0)),
            pl.BlockSpec((1, 1, hw), lambda h: (h, 0, 0)),
            pl.BlockSpec((1, D, hw), lambda h: (h, 0, 0)),
            pl.BlockSpec((1, 1, hw), lambda h: (h, 0, 0)),
            pl.BlockSpec((1, hw, D), lambda h: (h, 0, 0)),
            pl.BlockSpec((1, D), lambda h: (0, 0)),
            pl.BlockSpec((1, D), lambda h: (0, 0)),
            pl.BlockSpec((1, D), lambda h: (0, 0)),
        ],
        out_specs=pl.BlockSpec((Sq, D), lambda h: (0, 0)),
        out_shape=jax.ShapeDtypeStruct((Sq, D), jnp.float32),
        scratch_shapes=[pltpu.VMEM((Sq, D), jnp.float32)],
    )(
        xq, xkv,
        _heads_w(ap['wq']), ap['bq'].reshape(_NHS, 1, hw),
        _heads_w(ap['wk']), ap['bk'].reshape(_NHS, 1, hw),
        _heads_w(ap['wv']), ap['bv'].reshape(_NHS, 1, hw),
        ap['wo'].reshape(_NHS, hw, D), ap['bo'].reshape(1, D),
        lnp['g'].reshape(1, D), lnp['b'].reshape(1, D),
    )


def _tb_for(T):
    return 512 if T >= 2048 else 256
_RANK_CH = 512


def _router_body(x_ref, rw_ref, rb_ref, dest_ref, wc_ref, be_ref):
    x = x_ref[...]
    T = x.shape[0]
    logits = (
        jnp.dot(x, rw_ref[...], preferred_element_type=jnp.float32) + rb_ref[...]
    )
    mx = jnp.max(logits, axis=-1, keepdims=True)
    ex = jnp.exp(logits - mx)
    probs = ex / jnp.sum(ex, axis=-1, keepdims=True)
    cols = lax.broadcasted_iota(jnp.int32, probs.shape, 1)
    m1 = jnp.max(probs, axis=-1, keepdims=True)
    i1 = jnp.min(jnp.where(probs == m1, cols, NE), axis=-1, keepdims=True)
    masked = jnp.where(cols == i1, -1.0, probs)
    m2 = jnp.max(masked, axis=-1, keepdims=True)
    i2 = jnp.min(jnp.where(masked == m2, cols, NE), axis=-1, keepdims=True)
    s = m1 + m2
    wc_ref[...] = jnp.concatenate([m1 / s, m2 / s], axis=1)

    e_all = jnp.concatenate([i1, i2], axis=0)
    cols2 = lax.broadcasted_iota(jnp.int32, (2 * T, NE), 1)
    onehot = (e_all == cols2).astype(jnp.float32)

    rows_i = lax.broadcasted_iota(jnp.int32, (_RANK_CH, _RANK_CH), 0)
    cols_i = lax.broadcasted_iota(jnp.int32, (_RANK_CH, _RANK_CH), 1)
    tril = (rows_i > cols_i).astype(jnp.float32)
    ranks = []
    base = jnp.zeros((1, NE), jnp.float32)
    for c in range(2 * T // _RANK_CH):
        oh = onehot[c * _RANK_CH:(c + 1) * _RANK_CH]
        pc = jnp.dot(tril, oh, preferred_element_type=jnp.float32) + base
        ranks.append(jnp.sum(pc * oh, axis=1, keepdims=True))
        base = base + jnp.sum(oh, axis=0, keepdims=True)
    rank = jnp.concatenate(ranks, axis=0)
    cnt = base
    tb = _tb_for(T)
    padded = jnp.floor((cnt + (tb - 1)) * (1.0 / tb)).astype(jnp.int32) * tb
    er = lax.broadcasted_iota(jnp.int32, (NE, NE), 0)
    ec = lax.broadcasted_iota(jnp.int32, (NE, NE), 1)
    triu = (er < ec).astype(jnp.float32)
    off = jnp.dot(padded.astype(jnp.float32), triu,
                  preferred_element_type=jnp.float32)
    off_a = jnp.dot(onehot, off.reshape(NE, 1),
                    preferred_element_type=jnp.float32)
    dest_ref[...] = (off_a + rank + 0.5).astype(jnp.int32)

    NB = be_ref.shape[0]
    gstart = lax.broadcasted_iota(jnp.int32, (NB, NE), 0) * tb
    be_ref[...] = jnp.sum(
        (gstart >= off.astype(jnp.int32)).astype(jnp.int32),
        axis=1, keepdims=True) - 1


def _router(x, mp, nb):
    T = x.shape[0]
    return pl.pallas_call(
        _router_body,
        out_shape=(
            jax.ShapeDtypeStruct((2 * T, 1), jnp.int32),
            jax.ShapeDtypeStruct((T, 2), jnp.float32),
            jax.ShapeDtypeStruct((nb, 1), jnp.int32),
        ),
    )(x, mp['rw'], mp['rb'].reshape(1, NE))


def _sc_scatter_rows(x, dest, gtot):
    T = x.shape[0]
    ch = 2 * T // 32
    mesh = plsc.VectorSubcoreMesh(core_axis_name="c", subcore_axis_name="s", num_cores=2, num_subcores=16)

    def body(x_hbm, dest_hbm, out_hbm, idx_v, rows_v, sem):
        wid = lax.axis_index("s") * 2 + lax.axis_index("c")
        base = wid * ch
        tok = base % T
        pltpu.sync_copy(dest_hbm.at[pl.ds(base, ch)], idx_v)
        pltpu.sync_copy(x_hbm.at[pl.ds(tok, ch)], rows_v)
        pltpu.async_copy(rows_v, out_hbm.at[idx_v], sem).wait()

    f = functools.partial(
        pl.kernel, body, mesh=mesh,
        out_type=jax.ShapeDtypeStruct((gtot, D), jnp.float32),
        scratch_types=[
            pltpu.VMEM((ch,), jnp.int32),
            pltpu.VMEM((ch, D), jnp.float32),
            pltpu.SemaphoreType.DMA,
        ],
    )
    return f()(x, dest)


def _sc_gather_rows(sy, dest, ntot):
    ch = ntot // 32
    mesh = plsc.VectorSubcoreMesh(core_axis_name="c", subcore_axis_name="s", num_cores=2, num_subcores=16)

    def body(sy_hbm, dest_hbm, out_hbm, idx_v, rows_v, sem):
        wid = lax.axis_index("s") * 2 + lax.axis_index("c")
        base = wid * ch
        pltpu.sync_copy(dest_hbm.at[pl.ds(base, ch)], idx_v)
        pltpu.async_copy(sy_hbm.at[idx_v], rows_v, sem).wait()
        pltpu.sync_copy(rows_v, out_hbm.at[pl.ds(base, ch)])

    f = functools.partial(
        pl.kernel, body, mesh=mesh,
        out_type=jax.ShapeDtypeStruct((ntot, D), jnp.float32),
        scratch_types=[
            pltpu.VMEM((ch,), jnp.int32),
            pltpu.VMEM((ch, D), jnp.float32),
            pltpu.SemaphoreType.DMA,
        ],
    )
    return f()(sy, dest)


def _ffn_body(be_ref, x_ref, w1_ref, b1_ref, w2_ref, b2_ref, o_ref):
    del be_ref
    h = jnp.maximum(
        jnp.dot(x_ref[...], w1_ref[0], preferred_element_type=jnp.float32)
        + b1_ref[0],
        0.0,
    )
    o_ref[...] = (
        jnp.dot(h, w2_ref[0], preferred_element_type=jnp.float32) + b2_ref[0]
    )


def _grouped_ffn(sx, be, mp, nb, tb):
    spec = pltpu.PrefetchScalarGridSpec(
        num_scalar_prefetch=1,
        grid=(nb,),
        in_specs=[
            pl.BlockSpec((tb, D), lambda g, be: (g, 0)),
            pl.BlockSpec((1, D, DFF), lambda g, be: (be[g], 0, 0)),
            pl.BlockSpec((1, 1, DFF), lambda g, be: (be[g], 0, 0)),
            pl.BlockSpec((1, DFF, D), lambda g, be: (be[g], 0, 0)),
            pl.BlockSpec((1, 1, D), lambda g, be: (be[g], 0, 0)),
        ],
        out_specs=pl.BlockSpec((tb, D), lambda g, be: (g, 0)),
    )
    return pl.pallas_call(
        _ffn_body,
        grid_spec=spec,
        out_shape=jax.ShapeDtypeStruct((nb * tb, D), jnp.float32),
    )(be, sx, mp['w1'], mp['b1'].reshape(NE, 1, DFF), mp['w2'],
      mp['b2'].reshape(NE, 1, D))


def _combine_body(x_ref, y0_ref, y1_ref, wc_ref, g_ref, bb_ref, o_ref):
    wc = wc_ref[...]
    t = x_ref[...] + wc[:, 0:1] * y0_ref[...] + wc[:, 1:2] * y1_ref[...]
    mu = jnp.mean(t, axis=-1, keepdims=True)
    var = jnp.mean((t - mu) ** 2, axis=-1, keepdims=True)
    o_ref[...] = (t - mu) * jax.lax.rsqrt(var + 1e-5) * g_ref[...] + bb_ref[...]


def _combine_ln(x, yall, wc, g, beta):
    T = x.shape[0]
    return pl.pallas_call(
        _combine_body,
        out_shape=jax.ShapeDtypeStruct((T, D), jnp.float32),
    )(x, yall[:T], yall[T:], wc, g.reshape(1, D), beta.reshape(1, D))


def _moe_ln(x, mp, g, beta):
    T = x.shape[0]
    tb = _tb_for(T)
    nb = 2 * T // tb + NE
    dest, wc, be = _router(x, mp, nb)
    dest1 = dest.reshape(2 * T)
    sx = _sc_scatter_rows(x, dest1, nb * tb)
    sy = _grouped_ffn(sx, be.reshape(nb), mp, nb, tb)
    yall = _sc_gather_rows(sy, dest1, 2 * T)
    return _combine_ln(x, yall, wc, g, beta)


def kernel(src, tgt, params):
    p = params
    src_i = src[0]
    tgt_i = tgt[0]

    x = _embed(p['enc_emb'], src_i, _pe_table(src_i.shape[0]))
    for lp in p['enc']:
        x = _mha_ln(x, x, lp['sa'], lp['ln1'])
        x = _moe_ln(x, lp['moe'], lp['ln2']['g'], lp['ln2']['b'])
    mem = x

    y = _embed(p['dec_emb'], tgt_i, _pe_table(tgt_i.shape[0]))
    for lp in p['dec']:
        y = _mha_ln(y, y, lp['sa'], lp['ln1'])
        y = _mha_ln(y, mem, lp['ca'], lp['ln2'])
        y = _moe_ln(y, lp['moe'], lp['ln3']['g'], lp['ln3']['b'])

    logits = _mm_bias_blocked(y, p['out_w'], p['out_b'], 6400)
    return logits[None]


# speedup vs baseline: 1.1368x; 1.0118x over previous
import functools
import math

import jax
import jax.numpy as jnp
import numpy as np
from jax import lax
from jax.experimental import pallas as pl
from jax.experimental.pallas import tpu as pltpu
from jax.experimental.pallas import tpu_sc as plsc

V = 32000
D = 512
H = 8
NE = 8
DFF = 1024
DH = D // H
_SQRT_D = math.sqrt(D)
_ATTN_SCALE = 1.0 / math.sqrt(DH)
_EMB_ROWS = 16


def _pe_table(S):
    pos = np.arange(S)[:, None].astype(np.float32)
    div = np.exp(np.arange(0, D, 2).astype(np.float32) * (-math.log(10000.0) / D))
    pe = np.zeros((S, D), dtype=np.float32)
    pe[:, 0::2] = np.sin(pos * div)
    pe[:, 1::2] = np.cos(pos * div)
    return jnp.asarray(pe)


def _sc_row_gather(table, idx):
    T = idx.shape[0]
    ch = T // 32
    mesh = plsc.VectorSubcoreMesh(core_axis_name="c", subcore_axis_name="s",
                                  num_cores=2, num_subcores=16)

    def body(table_hbm, idx_hbm, out_hbm, idx_v, rows_v, sem):
        wid = lax.axis_index("s") * 2 + lax.axis_index("c")
        base = wid * ch
        pltpu.sync_copy(idx_hbm.at[pl.ds(base, ch)], idx_v)
        pltpu.async_copy(table_hbm.at[idx_v], rows_v, sem).wait()
        pltpu.sync_copy(rows_v, out_hbm.at[pl.ds(base, ch)])

    f = functools.partial(
        pl.kernel, body, mesh=mesh,
        out_type=jax.ShapeDtypeStruct((T, D), jnp.float32),
        scratch_types=[
            pltpu.VMEM((ch,), jnp.int32),
            pltpu.VMEM((ch, D), jnp.float32),
            pltpu.SemaphoreType.DMA,
        ],
    )
    return f()(table, idx)


def _scale_pe_body(x_ref, pe_ref, o_ref):
    o_ref[...] = x_ref[...] * _SQRT_D + pe_ref[...]


def _embed(table, idx, pe):
    T = idx.shape[0]
    rows = _sc_row_gather(table, idx)
    return pl.pallas_call(
        _scale_pe_body,
        out_shape=jax.ShapeDtypeStruct((T, D), jnp.float32),
    )(rows, pe)


def _mm_bias_body(a_ref, w_ref, b_ref, o_ref):
    o_ref[...] = (
        jnp.dot(a_ref[...], w_ref[...], preferred_element_type=jnp.float32)
        + b_ref[...]
    )


def _mm_bias(a, w, b):
    M, _ = a.shape
    N = w.shape[1]
    return pl.pallas_call(
        _mm_bias_body,
        out_shape=jax.ShapeDtypeStruct((M, N), jnp.float32),
    )(a, w, b.reshape(1, N))


def _mm_bias_blocked(a, w, b, nb):
    M, K = a.shape
    N = w.shape[1]
    return pl.pallas_call(
        _mm_bias_body,
        grid=(N // nb,),
        in_specs=[
            pl.BlockSpec((M, K), lambda j: (0, 0)),
            pl.BlockSpec((K, nb), lambda j: (0, j)),
            pl.BlockSpec((1, nb), lambda j: (0, j)),
        ],
        out_specs=pl.BlockSpec((M, nb), lambda j: (0, j)),
        out_shape=jax.ShapeDtypeStruct((M, N), jnp.float32),
    )(a, w, b.reshape(1, N))


_HP = 2
_NHS = H // _HP


def _attn_one_head(q, k, v):
    s = jax.lax.dot_general(
        q, k, (((1,), (1,)), ((), ())), preferred_element_type=jnp.float32
    ) * _ATTN_SCALE
    m = jnp.max(s, axis=-1, keepdims=True)
    p = jnp.exp(s - m)
    r = jnp.sum(p, axis=-1, keepdims=True)
    pb = p.astype(jnp.bfloat16)
    return jnp.dot(pb, v, preferred_element_type=jnp.float32) * (1.0 / r)


def _mha_body(xq_ref, xkv_ref, wq_ref, bq_ref, wk_ref, bk_ref, wv_ref, bv_ref,
              wo_ref, bo_ref, g_ref, bb_ref, o_ref, acc_ref):
    h = pl.program_id(0)
    xq = xq_ref[...]
    xkv = xkv_ref[...]
    q = jnp.dot(xq, wq_ref[0], preferred_element_type=jnp.float32) + bq_ref[0]
    k = jnp.dot(xkv, wk_ref[0], preferred_element_type=jnp.float32) + bk_ref[0]
    v = jnp.dot(xkv, wv_ref[0], preferred_element_type=jnp.float32) + bv_ref[0]
    oh = jnp.concatenate(
        [_attn_one_head(q[:, j * DH:(j + 1) * DH], k[:, j * DH:(j + 1) * DH],
                        v[:, j * DH:(j + 1) * DH]) for j in range(_HP)],
        axis=1,
    )
    contrib = jnp.dot(oh, wo_ref[0], preferred_element_type=jnp.float32)

    @pl.when(h == 0)
    def _():
        acc_ref[...] = contrib

    @pl.when(h != 0)
    def _():
        acc_ref[...] += contrib

    @pl.when(h == _NHS - 1)
    def _():
        t = acc_ref[...] + bo_ref[...] + xq
        mu = jnp.mean(t, axis=-1, keepdims=True)
        var = jnp.mean((t - mu) ** 2, axis=-1, keepdims=True)
        o_ref[...] = (t - mu) * jax.lax.rsqrt(var + 1e-5) * g_ref[...] + bb_ref[...]


def _heads_w(w):
    return w.reshape(D, _NHS, _HP * DH).transpose(1, 0, 2)


def _mha_ln(xq, xkv, ap, lnp):
    Sq = xq.shape[0]
    Skv = xkv.shape[0]
    hw = _HP * DH
    return pl.pallas_call(
        _mha_body,
        grid=(_NHS,),
        in_specs=[
            pl.BlockSpec((Sq, D), lambda h: (0, 0)),
            pl.BlockSpec((Skv, D), lambda h: (0, 0)),
            pl.BlockSpec((1, D, hw), lambda h: (h, 0, 0)),
            pl.BlockSpec((1, 1, hw), lambda h: (h, 0, 0)),
            pl.BlockSpec((1, D, hw), lambda h: (h, 0, 0)),
            pl.BlockSpec((1, 1, hw), lambda h: (h, 0, 0)),
            pl.BlockSpec((1, D, hw), lambda h: (h, 0, 0)),
            pl.BlockSpec((1, 1, hw), lambda h: (h, 0, 0)),
            pl.BlockSpec((1, hw, D), lambda h: (h, 0, 0)),
            pl.BlockSpec((1, D), lambda h: (0, 0)),
            pl.BlockSpec((1, D), lambda h: (0, 0)),
            pl.BlockSpec((1, D), lambda h: (0, 0)),
        ],
        out_specs=pl.BlockSpec((Sq, D), lambda h: (0, 0)),
        out_shape=jax.ShapeDtypeStruct((Sq, D), jnp.float32),
        scratch_shapes=[pltpu.VMEM((Sq, D), jnp.float32)],
    )(
        xq, xkv,
        _heads_w(ap['wq']), ap['bq'].reshape(_NHS, 1, hw),
        _heads_w(ap['wk']), ap['bk'].reshape(_NHS, 1, hw),
        _heads_w(ap['wv']), ap['bv'].reshape(_NHS, 1, hw),
        ap['wo'].reshape(_NHS, hw, D), ap['bo'].reshape(1, D),
        lnp['g'].reshape(1, D), lnp['b'].reshape(1, D),
    )


def _tb_for(T):
    return 512 if T >= 2048 else 256
_RANK_CH = 512


def _router_body(x_ref, rw_ref, rb_ref, dest_ref, wc_ref, be_ref, valid_ref):
    x = x_ref[...]
    T = x.shape[0]
    logits = (
        jnp.dot(x, rw_ref[...], preferred_element_type=jnp.float32) + rb_ref[...]
    )
    mx = jnp.max(logits, axis=-1, keepdims=True)
    ex = jnp.exp(logits - mx)
    probs = ex / jnp.sum(ex, axis=-1, keepdims=True)
    cols = lax.broadcasted_iota(jnp.int32, probs.shape, 1)
    m1 = jnp.max(probs, axis=-1, keepdims=True)
    i1 = jnp.min(jnp.where(probs == m1, cols, NE), axis=-1, keepdims=True)
    masked = jnp.where(cols == i1, -1.0, probs)
    m2 = jnp.max(masked, axis=-1, keepdims=True)
    i2 = jnp.min(jnp.where(masked == m2, cols, NE), axis=-1, keepdims=True)
    s = m1 + m2
    wc_ref[...] = jnp.concatenate([m1 / s, m2 / s], axis=1)

    e_all = jnp.concatenate([i1, i2], axis=0)
    cols2 = lax.broadcasted_iota(jnp.int32, (2 * T, NE), 1)
    onehot = (e_all == cols2).astype(jnp.float32)

    rows_i = lax.broadcasted_iota(jnp.int32, (_RANK_CH, _RANK_CH), 0)
    cols_i = lax.broadcasted_iota(jnp.int32, (_RANK_CH, _RANK_CH), 1)
    tril = (rows_i > cols_i).astype(jnp.float32)
    ranks = []
    base = jnp.zeros((1, NE), jnp.float32)
    for c in range(2 * T // _RANK_CH):
        oh = onehot[c * _RANK_CH:(c + 1) * _RANK_CH]
        pc = jnp.dot(tril, oh, preferred_element_type=jnp.float32) + base
        ranks.append(jnp.sum(pc * oh, axis=1, keepdims=True))
        base = base + jnp.sum(oh, axis=0, keepdims=True)
    rank = jnp.concatenate(ranks, axis=0)
    cnt = base
    tb = _tb_for(T)
    padded = jnp.floor((cnt + (tb - 1)) * (1.0 / tb)).astype(jnp.int32) * tb
    er = lax.broadcasted_iota(jnp.int32, (NE, NE), 0)
    ec = lax.broadcasted_iota(jnp.int32, (NE, NE), 1)
    triu = (er < ec).astype(jnp.float32)
    off = jnp.dot(padded.astype(jnp.float32), triu,
                  preferred_element_type=jnp.float32)
    off_a = jnp.dot(onehot, off.reshape(NE, 1),
                    preferred_element_type=jnp.float32)
    dest_ref[...] = (off_a + rank + 0.5).astype(jnp.int32)

    NB = be_ref.shape[0]
    gstart = lax.broadcasted_iota(jnp.int32, (NB, NE), 0) * tb
    be_ref[...] = jnp.sum(
        (gstart >= off.astype(jnp.int32)).astype(jnp.int32),
        axis=1, keepdims=True) - 1
    tot = jnp.sum(padded, axis=1, keepdims=True)
    valid_ref[...] = (gstart[:, 0:1] < tot).astype(jnp.int32)


def _router(x, mp, nb):
    T = x.shape[0]
    return pl.pallas_call(
        _router_body,
        out_shape=(
            jax.ShapeDtypeStruct((2 * T, 1), jnp.int32),
            jax.ShapeDtypeStruct((T, 2), jnp.float32),
            jax.ShapeDtypeStruct((nb, 1), jnp.int32),
            jax.ShapeDtypeStruct((nb, 1), jnp.int32),
        ),
    )(x, mp['rw'], mp['rb'].reshape(1, NE))


def _sc_scatter_rows(x, dest, gtot):
    T = x.shape[0]
    ch = 2 * T // 32
    mesh = plsc.VectorSubcoreMesh(core_axis_name="c", subcore_axis_name="s", num_cores=2, num_subcores=16)

    def body(x_hbm, dest_hbm, out_hbm, idx_v, rows_v, sem):
        wid = lax.axis_index("s") * 2 + lax.axis_index("c")
        base = wid * ch
        tok = base % T
        pltpu.sync_copy(dest_hbm.at[pl.ds(base, ch)], idx_v)
        pltpu.sync_copy(x_hbm.at[pl.ds(tok, ch)], rows_v)
        pltpu.async_copy(rows_v, out_hbm.at[idx_v], sem).wait()

    f = functools.partial(
        pl.kernel, body, mesh=mesh,
        out_type=jax.ShapeDtypeStruct((gtot, D), jnp.float32),
        scratch_types=[
            pltpu.VMEM((ch,), jnp.int32),
            pltpu.VMEM((ch, D), jnp.float32),
            pltpu.SemaphoreType.DMA,
        ],
    )
    return f()(x, dest)


def _sc_gather_rows(sy, dest, ntot):
    ch = ntot // 32
    mesh = plsc.VectorSubcoreMesh(core_axis_name="c", subcore_axis_name="s", num_cores=2, num_subcores=16)

    def body(sy_hbm, dest_hbm, out_hbm, idx_v, rows_v, sem):
        wid = lax.axis_index("s") * 2 + lax.axis_index("c")
        base = wid * ch
        pltpu.sync_copy(dest_hbm.at[pl.ds(base, ch)], idx_v)
        pltpu.async_copy(sy_hbm.at[idx_v], rows_v, sem).wait()
        pltpu.sync_copy(rows_v, out_hbm.at[pl.ds(base, ch)])

    f = functools.partial(
        pl.kernel, body, mesh=mesh,
        out_type=jax.ShapeDtypeStruct((ntot, D), jnp.float32),
        scratch_types=[
            pltpu.VMEM((ch,), jnp.int32),
            pltpu.VMEM((ch, D), jnp.float32),
            pltpu.SemaphoreType.DMA,
        ],
    )
    return f()(sy, dest)


def _ffn_body(be_ref, valid_ref, x_ref, w1_ref, b1_ref, w2_ref, b2_ref, o_ref):
    del be_ref
    g = pl.program_id(0)

    @pl.when(valid_ref[g] == 1)
    def _():
        h = jnp.maximum(
            jnp.dot(x_ref[...], w1_ref[0], preferred_element_type=jnp.float32)
            + b1_ref[0],
            0.0,
        )
        o_ref[...] = (
            jnp.dot(h, w2_ref[0], preferred_element_type=jnp.float32) + b2_ref[0]
        )


def _grouped_ffn(sx, be, valid, mp, nb, tb):
    spec = pltpu.PrefetchScalarGridSpec(
        num_scalar_prefetch=2,
        grid=(nb,),
        in_specs=[
            pl.BlockSpec((tb, D), lambda g, be, valid: (g, 0)),
            pl.BlockSpec((1, D, DFF), lambda g, be, valid: (be[g], 0, 0)),
            pl.BlockSpec((1, 1, DFF), lambda g, be, valid: (be[g], 0, 0)),
            pl.BlockSpec((1, DFF, D), lambda g, be, valid: (be[g], 0, 0)),
            pl.BlockSpec((1, 1, D), lambda g, be, valid: (be[g], 0, 0)),
        ],
        out_specs=pl.BlockSpec((tb, D), lambda g, be, valid: (g, 0)),
    )
    return pl.pallas_call(
        _ffn_body,
        grid_spec=spec,
        out_shape=jax.ShapeDtypeStruct((nb * tb, D), jnp.float32),
    )(be, valid, sx, mp['w1'], mp['b1'].reshape(NE, 1, DFF), mp['w2'],
      mp['b2'].reshape(NE, 1, D))


def _combine_body(x_ref, y0_ref, y1_ref, wc_ref, g_ref, bb_ref, o_ref):
    wc = wc_ref[...]
    t = x_ref[...] + wc[:, 0:1] * y0_ref[...] + wc[:, 1:2] * y1_ref[...]
    mu = jnp.mean(t, axis=-1, keepdims=True)
    var = jnp.mean((t - mu) ** 2, axis=-1, keepdims=True)
    o_ref[...] = (t - mu) * jax.lax.rsqrt(var + 1e-5) * g_ref[...] + bb_ref[...]


def _combine_ln(x, yall, wc, g, beta):
    T = x.shape[0]
    return pl.pallas_call(
        _combine_body,
        out_shape=jax.ShapeDtypeStruct((T, D), jnp.float32),
    )(x, yall[:T], yall[T:], wc, g.reshape(1, D), beta.reshape(1, D))


def _moe_ln(x, mp, g, beta):
    T = x.shape[0]
    tb = _tb_for(T)
    nb = 2 * T // tb + NE
    dest, wc, be, valid = _router(x, mp, nb)
    dest1 = dest.reshape(2 * T)
    sx = _sc_scatter_rows(x, dest1, nb * tb)
    sy = _grouped_ffn(sx, be.reshape(nb), valid.reshape(nb), mp, nb, tb)
    yall = _sc_gather_rows(sy, dest1, 2 * T)
    return _combine_ln(x, yall, wc, g, beta)


def kernel(src, tgt, params):
    p = params
    src_i = src[0]
    tgt_i = tgt[0]

    x = _embed(p['enc_emb'], src_i, _pe_table(src_i.shape[0]))
    for lp in p['enc']:
        x = _mha_ln(x, x, lp['sa'], lp['ln1'])
        x = _moe_ln(x, lp['moe'], lp['ln2']['g'], lp['ln2']['b'])
    mem = x

    y = _embed(p['dec_emb'], tgt_i, _pe_table(tgt_i.shape[0]))
    for lp in p['dec']:
        y = _mha_ln(y, y, lp['sa'], lp['ln1'])
        y = _mha_ln(y, mem, lp['ca'], lp['ln2'])
        y = _moe_ln(y, lp['moe'], lp['ln3']['g'], lp['ln3']['b'])

    logits = _mm_bias_blocked(y, p['out_w'], p['out_b'], 6400)
    return logits[None]
